# baseline (speedup 1.0000x reference)
import functools
import jax
import jax.numpy as jnp
from jax import lax
from jax.experimental import pallas as pl
from jax.experimental.pallas import tpu as pltpu
from jax.experimental.pallas import tpu_sc as plsc

R = 16
N = 131072
TWO_N = 2 * N
W = 2048
NWIN = N // W
HIST = 65536
CHUNK = 8192
TOTAL = R * TWO_N
SETTLE = 50000

_mesh = plsc.VectorSubcoreMesh(core_axis_name="c", subcore_axis_name="s")


def _keys_from_f32(x, tag_u32):
    b = lax.bitcast_convert_type(x, jnp.uint32)
    neg = (b >> 31) != 0
    k = jnp.where(neg, ~b, b | jnp.uint32(0x80000000))
    return (k & jnp.uint32(0xFFFFFFFE)) | tag_u32


def _hist_zero(hist):
    def body(j, carry):
        hist[pl.ds(j * 16, 16)] = jnp.zeros((16,), jnp.int32)
        return carry
    lax.fori_loop(0, HIST // 16, body, 0)


def _scan_hist(hist, pchunk, shared_hist, s, h, init):
    partner = s ^ 1

    def chunk_body(ch, carry):
        pltpu.sync_copy(hist.at[pl.ds(ch * CHUNK, CHUNK)], shared_hist.at[s])
        plsc.subcore_barrier()
        pltpu.sync_copy(shared_hist.at[partner], pchunk)

        def vec_body(j, carry2):
            own = hist[pl.ds(ch * CHUNK + j * 16, 16)]
            par = pchunk[pl.ds(j * 16, 16)]
            tot = own + par
            incl = plsc.cumsum(tot)
            base = incl - tot + carry2 + par * h
            hist[pl.ds(ch * CHUNK + j * 16, 16)] = base
            return carry2 + jnp.sum(tot)

        carry = lax.fori_loop(0, CHUNK // 16, vec_body, carry)
        plsc.subcore_barrier()
        return carry

    lax.fori_loop(0, HIST // CHUNK, chunk_body, init)


@functools.partial(
    pl.kernel,
    out_type=[
        jax.ShapeDtypeStruct((512,), jnp.float32),
        jax.ShapeDtypeStruct((TOTAL,), jnp.uint32),
        jax.ShapeDtypeStruct((TOTAL,), jnp.uint32),
    ],
    mesh=_mesh,
    compiler_params=pltpu.CompilerParams(needs_layout_passes=False),
    scratch_types=[
        pltpu.VMEM((HIST,), jnp.int32),
        pltpu.VMEM((CHUNK,), jnp.int32),
        pltpu.VMEM((W,), jnp.float32),
        pltpu.VMEM((W,), jnp.uint32),
        pltpu.VMEM((W,), jnp.int32),
        pltpu.VMEM_SHARED((16, CHUNK), jnp.int32),
        pltpu.SemaphoreType.DMA,
    ],
)
def _sc_sort_reduce(u_hbm, v_hbm, t_out, buf_a, buf_b,
                    hist, pchunk, fwin, kwin, iwin, shared_hist, sem):
    c = lax.axis_index("c")
    s = lax.axis_index("s")
    row = c * 8 + s // 2
    h = s % 2
    tag = lax.convert_element_type(h, jnp.uint32)
    row_base = row * TWO_N
    half_base = row_base + h * N
    iota = lax.iota(jnp.int32, 16)

    _hist_zero(hist)

    def p1_win(src_ref):
        def body(w, carry):
            pltpu.sync_copy(src_ref.at[pl.ds(row * N + w * W, W)], fwin)

            def vec_body(j, carry2):
                x = fwin[pl.ds(j * 16, 16)]
                k = _keys_from_f32(x, tag)
                kwin[pl.ds(j * 16, 16)] = k
                d = (k & jnp.uint32(0xFFFF)).astype(jnp.int32)
                cnt, last = plsc.scan_count(d)
                plsc.addupdate_scatter(hist, [d], cnt, mask=last)
                return carry2

            lax.fori_loop(0, W // 16, vec_body, 0)
            pltpu.sync_copy(kwin, buf_b.at[pl.ds(half_base + w * W, W)])
            return carry

        lax.fori_loop(0, NWIN, body, 0)

    @pl.when(h == 0)
    def _():
        p1_win(u_hbm)

    @pl.when(h == 1)
    def _():
        p1_win(v_hbm)

    _scan_hist(hist, pchunk, shared_hist, s, h, row_base)

    def p1s_body(w, carry):
        pltpu.sync_copy(buf_b.at[pl.ds(half_base + w * W, W)], kwin)

        def vec_body(j, carry2):
            k = kwin[pl.ds(j * 16, 16)]
            d = (k & jnp.uint32(0xFFFF)).astype(jnp.int32)
            cnt, last = plsc.scan_count(d)
            base = plsc.load_gather(hist, [d])
            iwin[pl.ds(j * 16, 16)] = base + cnt - 1
            plsc.addupdate_scatter(hist, [d], cnt, mask=last)
            return carry2

        lax.fori_loop(0, W // 16, vec_body, 0)
        pltpu.async_copy(kwin, buf_a.at[iwin], sem).wait()
        return carry

    lax.fori_loop(0, NWIN, p1s_body, 0)
    plsc.subcore_barrier()
    settle = lax.fori_loop(0, SETTLE, lambda i, a: a * 1664525 + i, s)
    iwin[pl.ds(0, 16)] = jnp.broadcast_to(settle, (16,))
    plsc.subcore_barrier()

    _hist_zero(hist)

    def p2h_body(w, carry):
        pltpu.sync_copy(buf_a.at[pl.ds(half_base + w * W, W)], kwin)

        def vec_body(j, carry2):
            k = kwin[pl.ds(j * 16, 16)]
            d = (k >> 16).astype(jnp.int32)
            t = (k & jnp.uint32(1)).astype(jnp.int32)
            d2 = (d << 1) | t
            cnt_all, last = plsc.scan_count(d)
            cnt_tag, _ = plsc.scan_count(d2)
            sgn = 1 - 2 * t
            insig = sgn * (2 * cnt_tag - cnt_all)
            plsc.addupdate_scatter(hist, [d], insig, mask=last)
            return carry2

        lax.fori_loop(0, W // 16, vec_body, 0)
        return carry

    lax.fori_loop(0, NWIN, p2h_body, 0)
    _scan_hist(hist, pchunk, shared_hist, s, h, 0)

    def red_win(w, acc):
        pltpu.sync_copy(buf_a.at[pl.ds(half_base + w * W, W)], kwin)

        def vec_body(j, acc2):
            k = kwin[pl.ds(j * 16, 16)]
            d = (k >> 16).astype(jnp.int32)
            t = (k & jnp.uint32(1)).astype(jnp.int32)
            d2 = (d << 1) | t
            cnt_all, last = plsc.scan_count(d)
            cnt_tag, _ = plsc.scan_count(d2)
            sgn = 1 - 2 * t
            pref = sgn * (2 * cnt_tag - cnt_all)
            g = plsc.load_gather(hist, [d])
            s_glob = (g + pref).astype(jnp.float32)
            plsc.addupdate_scatter(hist, [d], pref, mask=last)
            neg = (k >> 31) == 0
            b = jnp.where(neg, ~k, k ^ jnp.uint32(0x80000000))
            x = lax.bitcast_convert_type(b, jnp.float32)
            sig = lax.convert_element_type(sgn, jnp.float32)
            return acc2 + x * (1.0 - 2.0 * sig * s_glob)

        return lax.fori_loop(0, W // 16, vec_body, acc)

    acc = lax.fori_loop(0, NWIN, red_win, jnp.zeros((16,), jnp.float32))
    t_half = jnp.sum(acc)
    outvec = jnp.where(iota == 0, t_half, 0.0)
    fwin[pl.ds(0, 16)] = outvec
    wid = c * 16 + s
    pltpu.sync_copy(fwin.at[pl.ds(0, 16)], t_out.at[pl.ds(wid * 16, 16)])


def kernel(u_values, v_values):
    t_parts, _, _ = _sc_sort_reduce(u_values.reshape(-1), v_values.reshape(-1))
    a = t_parts.reshape(2, 8, 2, 16)
    t_row = a[:, :, 0, 0] + a[:, :, 1, 0]
    dist = jnp.sqrt(jnp.maximum(t_row, 0.0)) / N
    return dist.sum() / R


# speedup vs baseline: 1.0480x; 1.0480x over previous
import functools
import jax
import jax.numpy as jnp
from jax import lax
from jax.experimental import pallas as pl
from jax.experimental.pallas import tpu as pltpu
from jax.experimental.pallas import tpu_sc as plsc

R = 16
N = 131072
TWO_N = 2 * N
W = 2048
NWIN = N // W
HIST = 65536
CHUNK = 8192
TOTAL = R * TWO_N
SETTLE = 8000

_mesh = plsc.VectorSubcoreMesh(core_axis_name="c", subcore_axis_name="s")


def _keys_from_f32(x, tag_u32):
    b = lax.bitcast_convert_type(x, jnp.uint32)
    neg = (b >> 31) != 0
    k = jnp.where(neg, ~b, b | jnp.uint32(0x80000000))
    return (k & jnp.uint32(0xFFFFFFFE)) | tag_u32


def _hist_zero(hist):
    def body(j, carry):
        hist[pl.ds(j * 16, 16)] = jnp.zeros((16,), jnp.int32)
        return carry
    lax.fori_loop(0, HIST // 16, body, 0)


def _scan_hist(hist, pchunk, shared_hist, s, h, init):
    partner = s ^ 1

    def chunk_body(ch, carry):
        pltpu.sync_copy(hist.at[pl.ds(ch * CHUNK, CHUNK)], shared_hist.at[s])
        plsc.subcore_barrier()
        pltpu.sync_copy(shared_hist.at[partner], pchunk)

        def vec_body(j, carry2):
            own = hist[pl.ds(ch * CHUNK + j * 16, 16)]
            par = pchunk[pl.ds(j * 16, 16)]
            tot = own + par
            incl = plsc.cumsum(tot)
            base = incl - tot + carry2 + par * h
            hist[pl.ds(ch * CHUNK + j * 16, 16)] = base
            return carry2 + jnp.sum(tot)

        carry = lax.fori_loop(0, CHUNK // 16, vec_body, carry)
        plsc.subcore_barrier()
        return carry

    lax.fori_loop(0, HIST // CHUNK, chunk_body, init)


@functools.partial(
    pl.kernel,
    out_type=[
        jax.ShapeDtypeStruct((512,), jnp.float32),
        jax.ShapeDtypeStruct((TOTAL,), jnp.uint32),
        jax.ShapeDtypeStruct((TOTAL,), jnp.uint32),
    ],
    mesh=_mesh,
    compiler_params=pltpu.CompilerParams(needs_layout_passes=False),
    scratch_types=[
        pltpu.VMEM((HIST,), jnp.int32),
        pltpu.VMEM((CHUNK,), jnp.int32),
        pltpu.VMEM((W,), jnp.float32),
        pltpu.VMEM((W,), jnp.uint32),
        pltpu.VMEM((W,), jnp.int32),
        pltpu.VMEM_SHARED((16, CHUNK), jnp.int32),
        pltpu.SemaphoreType.DMA,
    ],
)
def _sc_sort_reduce(u_hbm, v_hbm, t_out, buf_a, buf_b,
                    hist, pchunk, fwin, kwin, iwin, shared_hist, sem):
    c = lax.axis_index("c")
    s = lax.axis_index("s")
    row = c * 8 + s // 2
    h = s % 2
    tag = lax.convert_element_type(h, jnp.uint32)
    row_base = row * TWO_N
    half_base = row_base + h * N
    iota = lax.iota(jnp.int32, 16)

    _hist_zero(hist)

    def p1_win(src_ref):
        def body(w, carry):
            pltpu.sync_copy(src_ref.at[pl.ds(row * N + w * W, W)], fwin)

            def vec_body(j, carry2):
                for jj in range(4):
                    o = (j * 4 + jj) * 16
                    x = fwin[pl.ds(o, 16)]
                    k = _keys_from_f32(x, tag)
                    kwin[pl.ds(o, 16)] = k
                    d = (k & jnp.uint32(0xFFFF)).astype(jnp.int32)
                    cnt, last = plsc.scan_count(d)
                    plsc.addupdate_scatter(hist, [d], cnt, mask=last)
                return carry2

            lax.fori_loop(0, W // 64, vec_body, 0)
            pltpu.sync_copy(kwin, buf_b.at[pl.ds(half_base + w * W, W)])
            return carry

        lax.fori_loop(0, NWIN, body, 0)

    @pl.when(h == 0)
    def _():
        p1_win(u_hbm)

    @pl.when(h == 1)
    def _():
        p1_win(v_hbm)

    _scan_hist(hist, pchunk, shared_hist, s, h, row_base)

    def p1s_body(w, carry):
        pltpu.sync_copy(buf_b.at[pl.ds(half_base + w * W, W)], kwin)

        def vec_body(j, carry2):
            for jj in range(4):
                o = (j * 4 + jj) * 16
                k = kwin[pl.ds(o, 16)]
                d = (k & jnp.uint32(0xFFFF)).astype(jnp.int32)
                cnt, last = plsc.scan_count(d)
                base = plsc.load_gather(hist, [d])
                iwin[pl.ds(o, 16)] = base + cnt - 1
                plsc.addupdate_scatter(hist, [d], cnt, mask=last)
            return carry2

        lax.fori_loop(0, W // 64, vec_body, 0)
        pltpu.async_copy(kwin, buf_a.at[iwin], sem).wait()
        return carry

    lax.fori_loop(0, NWIN, p1s_body, 0)
    plsc.subcore_barrier()
    settle = lax.fori_loop(0, SETTLE, lambda i, a: a * 1664525 + i, s)
    iwin[pl.ds(0, 16)] = jnp.broadcast_to(settle, (16,))
    plsc.subcore_barrier()

    _hist_zero(hist)

    def p2h_body(w, carry):
        pltpu.sync_copy(buf_a.at[pl.ds(half_base + w * W, W)], kwin)

        def vec_body(j, carry2):
            for jj in range(4):
                o = (j * 4 + jj) * 16
                k = kwin[pl.ds(o, 16)]
                d = (k >> 16).astype(jnp.int32)
                t = (k & jnp.uint32(1)).astype(jnp.int32)
                d2 = (d << 1) | t
                cnt_all, last = plsc.scan_count(d)
                cnt_tag, _ = plsc.scan_count(d2)
                sgn = 1 - 2 * t
                insig = sgn * (2 * cnt_tag - cnt_all)
                plsc.addupdate_scatter(hist, [d], insig, mask=last)
            return carry2

        lax.fori_loop(0, W // 64, vec_body, 0)
        return carry

    lax.fori_loop(0, NWIN, p2h_body, 0)
    _scan_hist(hist, pchunk, shared_hist, s, h, 0)

    def red_win(w, acc):
        pltpu.sync_copy(buf_a.at[pl.ds(half_base + w * W, W)], kwin)

        def vec_body(j, acc2):
            for jj in range(4):
                o = (j * 4 + jj) * 16
                k = kwin[pl.ds(o, 16)]
                d = (k >> 16).astype(jnp.int32)
                t = (k & jnp.uint32(1)).astype(jnp.int32)
                d2 = (d << 1) | t
                cnt_all, last = plsc.scan_count(d)
                cnt_tag, _ = plsc.scan_count(d2)
                sgn = 1 - 2 * t
                pref = sgn * (2 * cnt_tag - cnt_all)
                g = plsc.load_gather(hist, [d])
                s_glob = (g + pref).astype(jnp.float32)
                plsc.addupdate_scatter(hist, [d], pref, mask=last)
                neg = (k >> 31) == 0
                b = jnp.where(neg, ~k, k ^ jnp.uint32(0x80000000))
                x = lax.bitcast_convert_type(b, jnp.float32)
                sig = lax.convert_element_type(sgn, jnp.float32)
                acc2 = acc2 + x * (1.0 - 2.0 * sig * s_glob)
            return acc2

        return lax.fori_loop(0, W // 64, vec_body, acc)

    acc = lax.fori_loop(0, NWIN, red_win, jnp.zeros((16,), jnp.float32))
    t_half = jnp.sum(acc)
    outvec = jnp.where(iota == 0, t_half, 0.0)
    fwin[pl.ds(0, 16)] = outvec
    wid = c * 16 + s
    pltpu.sync_copy(fwin.at[pl.ds(0, 16)], t_out.at[pl.ds(wid * 16, 16)])


def kernel(u_values, v_values):
    t_parts, _, _ = _sc_sort_reduce(u_values.reshape(-1), v_values.reshape(-1))
    a = t_parts.reshape(2, 8, 2, 16)
    t_row = a[:, :, 0, 0] + a[:, :, 1, 0]
    dist = jnp.sqrt(jnp.maximum(t_row, 0.0)) / N
    return dist.sum() / R
